# Initial kernel scaffold; baseline (speedup 1.0000x reference)
import jax
import jax.numpy as jnp
from jax.experimental import pallas as pl


def kernel(x, weight, bias, mask):
    raise NotImplementedError("write your pallas kernel here")


# speedup vs baseline: 3.7054x; 3.7054x over previous
import jax
import jax.numpy as jnp
from jax.experimental import pallas as pl
from jax.experimental.pallas import tpu as pltpu


def _round_up(a, m):
    return ((a + m - 1) // m) * m


def _matmul_bias_kernel(x_ref, w_ref, b_ref, o_ref):
    acc = jax.lax.dot_general(
        x_ref[...], w_ref[...],
        dimension_numbers=(((1,), (1,)), ((), ())),
        preferred_element_type=jnp.float32,
    )
    o_ref[...] = acc + b_ref[...]


def kernel(x, weight, bias, mask):
    B, K = x.shape
    V = weight.shape[0]
    out_dtype = x.dtype

    w_bf = (weight * mask[:, None]).astype(jnp.bfloat16)
    x_bf = x.astype(jnp.bfloat16)
    b_m = (bias * mask).astype(jnp.float32)[None, :]

    tb = min(1024, _round_up(B, 8))
    tn = min(1024, _round_up(V, 128))
    Bp, Vp, Kp = _round_up(B, tb), _round_up(V, tn), _round_up(K, 128)
    if Bp != B or Kp != K:
        x_bf = jnp.pad(x_bf, ((0, Bp - B), (0, Kp - K)))
    if Vp != V or Kp != K:
        w_bf = jnp.pad(w_bf, ((0, Vp - V), (0, Kp - K)))
    if Vp != V:
        b_m = jnp.pad(b_m, ((0, 0), (0, Vp - V)))

    grid = (Bp // tb, Vp // tn)
    out = pl.pallas_call(
        _matmul_bias_kernel,
        out_shape=jax.ShapeDtypeStruct((Bp, Vp), jnp.float32),
        grid=grid,
        in_specs=[
            pl.BlockSpec((tb, Kp), lambda i, j: (i, 0)),
            pl.BlockSpec((tn, Kp), lambda i, j: (j, 0)),
            pl.BlockSpec((1, tn), lambda i, j: (0, j)),
        ],
        out_specs=pl.BlockSpec((tb, tn), lambda i, j: (i, j)),
        compiler_params=pltpu.CompilerParams(
            dimension_semantics=("parallel", "parallel")),
    )(x_bf, w_bf, b_m)

    if Bp != B or Vp != V:
        out = out[:B, :V]
    return out.astype(out_dtype)


# speedup vs baseline: 4.7486x; 1.2816x over previous
import jax
import jax.numpy as jnp
from jax.experimental import pallas as pl
from jax.experimental.pallas import tpu as pltpu


def _round_up(a, m):
    return ((a + m - 1) // m) * m


def _matmul_bias_kernel(x_ref, w_ref, b_ref, o_ref):
    xb = x_ref[...].astype(jnp.bfloat16)
    acc = jax.lax.dot_general(
        xb, w_ref[...],
        dimension_numbers=(((1,), (1,)), ((), ())),
        preferred_element_type=jnp.float32,
    )
    o_ref[...] = acc + b_ref[...]


def kernel(x, weight, bias, mask):
    B, K = x.shape
    V = weight.shape[0]
    out_dtype = x.dtype

    w_bf = (weight * mask[:, None]).astype(jnp.bfloat16)
    b_m = (bias * mask).astype(jnp.float32)[None, :]

    tb = min(1024, _round_up(B, 8))
    Bp, Vp, Kp = _round_up(B, tb), _round_up(V, 128), _round_up(K, 128)
    if Bp != B or Kp != K:
        x = jnp.pad(x, ((0, Bp - B), (0, Kp - K)))
    if Vp != V or Kp != K:
        w_bf = jnp.pad(w_bf, ((0, Vp - V), (0, Kp - K)))
    if Vp != V:
        b_m = jnp.pad(b_m, ((0, 0), (0, Vp - V)))

    grid = (Bp // tb,)
    out = pl.pallas_call(
        _matmul_bias_kernel,
        out_shape=jax.ShapeDtypeStruct((Bp, Vp), jnp.float32),
        grid=grid,
        in_specs=[
            pl.BlockSpec((tb, Kp), lambda i: (i, 0)),
            pl.BlockSpec((Vp, Kp), lambda i: (0, 0)),
            pl.BlockSpec((1, Vp), lambda i: (0, 0)),
        ],
        out_specs=pl.BlockSpec((tb, Vp), lambda i: (i, 0)),
        compiler_params=pltpu.CompilerParams(
            dimension_semantics=("parallel",)),
    )(x, w_bf, b_m)

    if Bp != B or Vp != V:
        out = out[:B, :V]
    return out.astype(out_dtype)


# speedup vs baseline: 4.8553x; 1.0225x over previous
import jax
import jax.numpy as jnp
from jax.experimental import pallas as pl
from jax.experimental.pallas import tpu as pltpu


def _round_up(a, m):
    return ((a + m - 1) // m) * m


def _matmul_bias_kernel(x_ref, w_ref, b_ref, o_ref):
    xb = x_ref[...].astype(jnp.bfloat16)
    acc = jax.lax.dot_general(
        xb, w_ref[...],
        dimension_numbers=(((1,), (1,)), ((), ())),
        preferred_element_type=jnp.float32,
    )
    o_ref[...] = acc + b_ref[...]


def kernel(x, weight, bias, mask):
    B, K = x.shape
    V = weight.shape[0]
    out_dtype = x.dtype

    w_bf = (weight * mask[:, None]).astype(jnp.bfloat16)
    b_m = (bias * mask).astype(jnp.float32)[None, :]

    tb = min(512, _round_up(B, 8))
    Bp, Vp, Kp = _round_up(B, tb), _round_up(V, 128), _round_up(K, 128)
    if Bp != B or Kp != K:
        x = jnp.pad(x, ((0, Bp - B), (0, Kp - K)))
    if Vp != V or Kp != K:
        w_bf = jnp.pad(w_bf, ((0, Vp - V), (0, Kp - K)))
    if Vp != V:
        b_m = jnp.pad(b_m, ((0, 0), (0, Vp - V)))

    grid = (Bp // tb,)
    out = pl.pallas_call(
        _matmul_bias_kernel,
        out_shape=jax.ShapeDtypeStruct((Bp, Vp), jnp.float32),
        grid=grid,
        in_specs=[
            pl.BlockSpec((tb, Kp), lambda i: (i, 0)),
            pl.BlockSpec((Vp, Kp), lambda i: (0, 0)),
            pl.BlockSpec((1, Vp), lambda i: (0, 0)),
        ],
        out_specs=pl.BlockSpec((tb, Vp), lambda i: (i, 0)),
        compiler_params=pltpu.CompilerParams(
            dimension_semantics=("parallel",)),
    )(x, w_bf, b_m)

    if Bp != B or Vp != V:
        out = out[:B, :V]
    return out.astype(out_dtype)
